# baseline (speedup 1.0000x reference)
import functools

import jax
import jax.numpy as jnp
from jax import lax
from jax.experimental import pallas as pl
from jax.experimental.pallas import tpu as pltpu
from jax.experimental.pallas import tpu_sc as plsc

_L = 16


def _make_sc_add(B, R, Cg, C):
    half = C // 2
    mesh = plsc.VectorSubcoreMesh(core_axis_name="c", subcore_axis_name="s")

    @functools.partial(
        pl.kernel,
        mesh=mesh,
        out_type=jax.ShapeDtypeStruct((B * R * Cg, C), jnp.float32),
        scratch_types=[
            pltpu.VMEM((R, half), jnp.float32),
            pltpu.VMEM((Cg, half), jnp.float32),
            pltpu.VMEM((Cg, C), jnp.float32),
        ],
    )
    def sc_add(x_hbm, row_hbm, col_hbm, out_hbm, row_v, col_v, xv):
        w = lax.axis_index("s") * 2 + lax.axis_index("c")
        pltpu.sync_copy(row_hbm, row_v)
        pltpu.sync_copy(col_hbm, col_v)
        base = w * (R * Cg)

        def body(r, carry):
            off = base + r * Cg
            pltpu.sync_copy(x_hbm.at[pl.ds(off, Cg)], xv)
            for i in range(Cg):
                for j in range(half // _L):
                    s = _L * j
                    xv[i, s:s + _L] = xv[i, s:s + _L] + row_v[r, s:s + _L]
                    s2 = half + s
                    xv[i, s2:s2 + _L] = xv[i, s2:s2 + _L] + col_v[i, s:s + _L]
            pltpu.sync_copy(xv, out_hbm.at[pl.ds(off, Cg)])
            return carry

        lax.fori_loop(0, R, body, 0)

    return sc_add


def kernel(inputs, row_emb, col_emb):
    B, R, Cg, C = inputs.shape
    x2 = inputs.reshape(B * R * Cg, C)
    out = _make_sc_add(B, R, Cg, C)(x2, row_emb, col_emb)
    return out.reshape(B, R * Cg, C)
